# baseline (speedup 1.0000x reference)
import math
import functools

import jax
import jax.numpy as jnp
import numpy as np
from jax import lax
from jax.experimental import pallas as pl
from jax.experimental.pallas import tpu as pltpu
from jax.experimental.pallas import tpu_sc as plsc

DIM = 64
MAX_LEN = 5000
SQRT_DIM = math.sqrt(DIM)

LANES = 16
NWORKERS = 32
SBLK = 128
NBUF = 2

VOCAB = 1000000
PITCH = 136
TBLK = 128
N_FULL_BLK = VOCAB // TBLK
N_MAIN = (N_FULL_BLK // NWORKERS) * NWORKERS
MAIN_PER_W = N_MAIN // NWORKERS
N_EXTRA = N_FULL_BLK - N_MAIN
TAIL0 = N_FULL_BLK * TBLK
TAILC = VOCAB - TAIL0


def _make_pe_t(seq_len: int) -> np.ndarray:
    position = np.arange(0, MAX_LEN, dtype=np.float64)[:, None]
    div_term = np.exp(
        np.arange(0, DIM, 2, dtype=np.float64) * -(math.log(10000.0) / DIM)
    )
    pe = np.zeros((MAX_LEN, DIM), dtype=np.float64)
    pe[:, 0::2] = np.sin(position * div_term)
    pe[:, 1::2] = np.cos(position * div_term)
    return np.ascontiguousarray(pe[:seq_len].T).astype(np.float32)


def _mesh():
    return plsc.VectorSubcoreMesh(core_axis_name="core",
                                  subcore_axis_name="subcore")


_SC_PARAMS = pltpu.CompilerParams(use_tc_tiling_on_sc=True,
                                  needs_layout_passes=False)


def _worker_id():
    return lax.axis_index("core") * 16 + lax.axis_index("subcore")


def _transpose_block(in_ref, out_ref, cols, unrolled):
    iot = jax.lax.iota(jnp.int32, LANES)
    parbase = lax.shift_left(lax.bitwise_and(iot, 1), 6)
    rowvs = [lax.shift_right_logical(iot, 1) + 8 * k
             for k in range(cols // LANES)]

    def one_row(d):
        colv = parbase + d
        vals = [in_ref[d, pl.ds(k * LANES, LANES)]
                for k in range(cols // LANES)]
        for k in range(cols // LANES):
            plsc.store_scatter(out_ref, [rowvs[k], colv], vals[k])

    if unrolled:
        for d in range(DIM):
            one_row(d)
    else:
        @pl.loop(0, DIM)
        def _(d):
            one_row(d)


@functools.partial(jax.jit, static_argnames=("S", "B"))
def _embed_sc(idx_t, W_t, pe_t, *, S, B):
    n_sblk = S // SBLK
    b_half = B * n_sblk // NWORKERS
    n_groups = SBLK // LANES

    @pl.kernel(
        out_type=jax.ShapeDtypeStruct((VOCAB // 2, 2 * DIM), jnp.float32),
        mesh=_mesh(),
        compiler_params=_SC_PARAMS,
        scratch_types=[
            pltpu.VMEM((NBUF, DIM, TBLK), jnp.float32),
            pltpu.VMEM((NBUF, TBLK // 2, PITCH), jnp.float32),
            pltpu.VMEM((DIM, TAILC), jnp.float32),
            pltpu.VMEM((TAILC // 2, PITCH), jnp.float32),
            pltpu.SemaphoreType.DMA((NBUF,)),
            pltpu.SemaphoreType.DMA((NBUF,)),
        ],
    )
    def transpose_fn(Wt_hbm, W2_hbm, in_v, out_v, tin_v, tout_v,
                     sem_i, sem_o):
        w = _worker_id()

        def in_copy(k, slot):
            c0 = (w * MAIN_PER_W + k) * TBLK
            return pltpu.make_async_copy(
                Wt_hbm.at[:, pl.ds(c0, TBLK)], in_v.at[slot], sem_i.at[slot])

        def out_copy(k, slot):
            r0 = (w * MAIN_PER_W + k) * (TBLK // 2)
            return pltpu.make_async_copy(
                out_v.at[slot, :, pl.ds(0, 2 * DIM)],
                W2_hbm.at[pl.ds(r0, TBLK // 2)],
                sem_o.at[slot])

        in_copy(0, 0).start()

        @pl.loop(0, MAIN_PER_W, step=NBUF)
        def _(k0):
            for u in range(NBUF):
                k = k0 + u
                slot = u
                nslot = (u + 1) % NBUF

                @pl.when(k + 1 < MAIN_PER_W)
                def _(k=k, nslot=nslot):
                    in_copy(k + 1, nslot).start()

                in_copy(k, slot).wait()

                @pl.when(k >= NBUF)
                def _(k=k, slot=slot):
                    out_copy(k - NBUF, slot).wait()

                _transpose_block(in_v.at[slot], out_v.at[slot], TBLK,
                                 unrolled=True)
                out_copy(k, slot).start()

        for u in range(NBUF):
            out_copy(MAIN_PER_W - NBUF + u, u).wait()

        @pl.when(w < N_EXTRA)
        def _():
            c0 = (N_MAIN + w) * TBLK
            pltpu.async_copy(Wt_hbm.at[:, pl.ds(c0, TBLK)], in_v.at[0],
                             sem_i.at[0]).wait()
            _transpose_block(in_v.at[0], out_v.at[0], TBLK, unrolled=False)
            pltpu.async_copy(out_v.at[0, :, pl.ds(0, 2 * DIM)],
                             W2_hbm.at[pl.ds((N_MAIN + w) * (TBLK // 2),
                                             TBLK // 2)],
                             sem_o.at[0]).wait()

        @pl.when(w == N_EXTRA)
        def _():
            pltpu.async_copy(Wt_hbm.at[:, pl.ds(TAIL0, TAILC)], tin_v,
                             sem_i.at[0]).wait()
            _transpose_block(tin_v, tout_v, TAILC, unrolled=False)
            pltpu.async_copy(tout_v.at[:, pl.ds(0, 2 * DIM)],
                             W2_hbm.at[pl.ds(TAIL0 // 2, TAILC // 2)],
                             sem_o.at[0]).wait()

    W2 = transpose_fn(W_t)

    @pl.kernel(
        out_type=jax.ShapeDtypeStruct((B, DIM, S), jnp.float32),
        mesh=_mesh(),
        compiler_params=_SC_PARAMS,
        scratch_types=[
            pltpu.VMEM((b_half, SBLK), jnp.int32),
            pltpu.VMEM((DIM, SBLK), jnp.float32),
            pltpu.VMEM((NBUF, SBLK), jnp.int32),
            pltpu.VMEM((NBUF, SBLK, PITCH), jnp.float32),
            pltpu.VMEM((NBUF, DIM, SBLK), jnp.float32),
            pltpu.SemaphoreType.DMA,
            pltpu.SemaphoreType.DMA((NBUF,)),
            pltpu.SemaphoreType.DMA((NBUF,)),
        ],
    )
    def gather_fn(W2_hbm, i_hbm, pe_hbm, o_hbm,
                  idx_v, pe_v, idxp_v, bufp_v, out_v,
                  sem_in, sem_g, sem_s):
        w = _worker_id()
        sblk = w // 2
        b0 = (w % 2) * b_half
        s0 = sblk * SBLK

        c_idx = pltpu.async_copy(
            i_hbm.at[pl.ds(b0, b_half), pl.ds(s0, SBLK)], idx_v, sem_in)
        c_pe = pltpu.async_copy(pe_hbm.at[:, pl.ds(s0, SBLK)], pe_v, sem_in)
        c_idx.wait()
        c_pe.wait()

        def prep_idx(c, slot):
            for g in range(n_groups):
                sl = pl.ds(g * LANES, LANES)
                idxp_v[slot, sl] = lax.shift_right_logical(idx_v[c, sl], 1)

        def gather_copy(slot):
            return pltpu.make_async_copy(
                W2_hbm.at[idxp_v.at[slot]],
                bufp_v.at[slot, :, pl.ds(0, SBLK)],
                sem_g.at[slot])

        def compute(c, slot):
            rowvs = [jax.lax.iota(jnp.int32, LANES) + g * LANES
                     for g in range(n_groups)]
            par64s = [lax.shift_left(
                lax.bitwise_and(idx_v[c, pl.ds(g * LANES, LANES)], 1), 6)
                for g in range(n_groups)]

            @pl.loop(0, DIM, step=8)
            def _(d0):
                for u in range(8):
                    d = d0 + u
                    vals = [plsc.load_gather(
                        bufp_v.at[slot], [rowvs[g], par64s[g] + d])
                        for g in range(n_groups)]
                    pes = [pe_v[d, pl.ds(g * LANES, LANES)]
                           for g in range(n_groups)]
                    for g in range(n_groups):
                        out_v[slot, d, pl.ds(g * LANES, LANES)] = (
                            vals[g] * SQRT_DIM + pes[g])

        def writeback_copy(c, slot):
            return pltpu.make_async_copy(
                out_v.at[slot],
                o_hbm.at[b0 + c, :, pl.ds(s0, SBLK)],
                sem_s.at[slot])

        prep_idx(0, 0)
        gather_copy(0).start()

        @pl.loop(0, b_half, step=NBUF)
        def _(c0):
            for u in range(NBUF):
                c = c0 + u
                slot = u
                nslot = (u + 1) % NBUF

                @pl.when(c + 1 < b_half)
                def _(c=c, nslot=nslot):
                    prep_idx(c + 1, nslot)
                    gather_copy(nslot).start()

                gather_copy(slot).wait()

                @pl.when(c >= NBUF)
                def _(c=c, slot=slot):
                    writeback_copy(c - NBUF, slot).wait()

                compute(c, slot)
                writeback_copy(c, slot).start()

        for u in range(NBUF):
            writeback_copy(b_half - NBUF + u, u).wait()

    return gather_fn(W2, idx_t, pe_t)


def kernel(input, W):
    S, B, _ = input.shape
    idx_t = jnp.transpose(input[..., 0])
    W_t = jnp.transpose(W)
    pe_t = jnp.asarray(_make_pe_t(S))
    out_t = _embed_sc(idx_t, W_t, pe_t, S=S, B=B)
    return jnp.transpose(out_t, (2, 0, 1))


# speedup vs baseline: 1.0762x; 1.0762x over previous
import math
import functools

import jax
import jax.numpy as jnp
import numpy as np
from jax import lax
from jax.experimental import pallas as pl
from jax.experimental.pallas import tpu as pltpu
from jax.experimental.pallas import tpu_sc as plsc

DIM = 64
MAX_LEN = 5000
SQRT_DIM = math.sqrt(DIM)

LANES = 16
NWORKERS = 32
SBLK = 128
NBUF = 2

VOCAB = 1000000
PITCH = 134
TBLK = 128
N_FULL_BLK = VOCAB // TBLK
N_MAIN = (N_FULL_BLK // NWORKERS) * NWORKERS
MAIN_PER_W = N_MAIN // NWORKERS
N_EXTRA = N_FULL_BLK - N_MAIN
TAIL0 = N_FULL_BLK * TBLK
TAILC = VOCAB - TAIL0


def _make_pe_t(seq_len: int) -> np.ndarray:
    position = np.arange(0, MAX_LEN, dtype=np.float64)[:, None]
    div_term = np.exp(
        np.arange(0, DIM, 2, dtype=np.float64) * -(math.log(10000.0) / DIM)
    )
    pe = np.zeros((MAX_LEN, DIM), dtype=np.float64)
    pe[:, 0::2] = np.sin(position * div_term)
    pe[:, 1::2] = np.cos(position * div_term)
    return np.ascontiguousarray(pe[:seq_len].T).astype(np.float32)


def _mesh():
    return plsc.VectorSubcoreMesh(core_axis_name="core",
                                  subcore_axis_name="subcore")


_SC_PARAMS = pltpu.CompilerParams(use_tc_tiling_on_sc=True,
                                  needs_layout_passes=False)


def _worker_id():
    return lax.axis_index("core") * 16 + lax.axis_index("subcore")


def _transpose_block(in_ref, out_ref, cols, unrolled):
    iot = jax.lax.iota(jnp.int32, LANES)
    parbase = lax.shift_left(lax.bitwise_and(iot, 1), 6)
    rowvs = [lax.shift_right_logical(iot, 1) + 8 * k
             for k in range(cols // LANES)]

    def one_row(d):
        colv = parbase + d
        vals = [in_ref[d, pl.ds(k * LANES, LANES)]
                for k in range(cols // LANES)]
        for k in range(cols // LANES):
            plsc.store_scatter(out_ref, [rowvs[k], colv], vals[k])

    if unrolled:
        for d in range(DIM):
            one_row(d)
    else:
        @pl.loop(0, DIM)
        def _(d):
            one_row(d)


@functools.partial(jax.jit, static_argnames=("S", "B"))
def _embed_sc(idxp_t, off_t, W_t, pe2d, *, S, B):
    n_sblk = S // SBLK
    b_half = B * n_sblk // NWORKERS
    n_groups = SBLK // LANES

    @pl.kernel(
        out_type=jax.ShapeDtypeStruct((VOCAB // 2, 2 * DIM), jnp.float32),
        mesh=_mesh(),
        compiler_params=_SC_PARAMS,
        scratch_types=[
            pltpu.VMEM((NBUF, DIM, TBLK), jnp.float32),
            pltpu.VMEM((NBUF, TBLK // 2, PITCH), jnp.float32),
            pltpu.VMEM((DIM, TAILC), jnp.float32),
            pltpu.VMEM((TAILC // 2, PITCH), jnp.float32),
            pltpu.SemaphoreType.DMA((NBUF,)),
            pltpu.SemaphoreType.DMA((NBUF,)),
        ],
    )
    def transpose_fn(Wt_hbm, W2_hbm, in_v, out_v, tin_v, tout_v,
                     sem_i, sem_o):
        w = _worker_id()

        def in_copy(k, slot):
            c0 = (w * MAIN_PER_W + k) * TBLK
            return pltpu.make_async_copy(
                Wt_hbm.at[:, pl.ds(c0, TBLK)], in_v.at[slot], sem_i.at[slot])

        def out_copy(k, slot):
            r0 = (w * MAIN_PER_W + k) * (TBLK // 2)
            return pltpu.make_async_copy(
                out_v.at[slot, :, pl.ds(0, 2 * DIM)],
                W2_hbm.at[pl.ds(r0, TBLK // 2)],
                sem_o.at[slot])

        in_copy(0, 0).start()

        @pl.loop(0, MAIN_PER_W, step=NBUF)
        def _(k0):
            for u in range(NBUF):
                k = k0 + u
                slot = u
                nslot = (u + 1) % NBUF

                @pl.when(k + 1 < MAIN_PER_W)
                def _(k=k, nslot=nslot):
                    in_copy(k + 1, nslot).start()

                in_copy(k, slot).wait()

                @pl.when(k >= NBUF)
                def _(k=k, slot=slot):
                    out_copy(k - NBUF, slot).wait()

                _transpose_block(in_v.at[slot], out_v.at[slot], TBLK,
                                 unrolled=True)
                out_copy(k, slot).start()

        for u in range(NBUF):
            out_copy(MAIN_PER_W - NBUF + u, u).wait()

        @pl.when(w < N_EXTRA)
        def _():
            c0 = (N_MAIN + w) * TBLK
            pltpu.async_copy(Wt_hbm.at[:, pl.ds(c0, TBLK)], in_v.at[0],
                             sem_i.at[0]).wait()
            _transpose_block(in_v.at[0], out_v.at[0], TBLK, unrolled=False)
            pltpu.async_copy(out_v.at[0, :, pl.ds(0, 2 * DIM)],
                             W2_hbm.at[pl.ds((N_MAIN + w) * (TBLK // 2),
                                             TBLK // 2)],
                             sem_o.at[0]).wait()

        @pl.when(w == N_EXTRA)
        def _():
            pltpu.async_copy(Wt_hbm.at[:, pl.ds(TAIL0, TAILC)], tin_v,
                             sem_i.at[0]).wait()
            _transpose_block(tin_v, tout_v, TAILC, unrolled=False)
            pltpu.async_copy(tout_v.at[:, pl.ds(0, 2 * DIM)],
                             W2_hbm.at[pl.ds(TAIL0 // 2, TAILC // 2)],
                             sem_o.at[0]).wait()

    W2 = transpose_fn(W_t)

    @pl.kernel(
        out_type=jax.ShapeDtypeStruct((B, S, DIM), jnp.float32),
        mesh=_mesh(),
        compiler_params=_SC_PARAMS,
        scratch_types=[
            pltpu.VMEM((b_half, SBLK), jnp.int32),
            pltpu.VMEM((SBLK, DIM), jnp.float32),
            pltpu.VMEM((NBUF, SBLK), jnp.int32),
            pltpu.VMEM((NBUF, SBLK, SBLK), jnp.float32),
            pltpu.VMEM((NBUF, SBLK, DIM), jnp.float32),
            pltpu.SemaphoreType.DMA,
            pltpu.SemaphoreType.DMA((NBUF,)),
            pltpu.SemaphoreType.DMA((NBUF,)),
            pltpu.SemaphoreType.DMA((NBUF,)),
        ],
    )
    def gather_fn(W2_hbm, ip_hbm, off_hbm, pe_hbm, o_hbm,
                  idxp_v, pe_v, off_s, buf_v, out_v,
                  sem_in, sem_g, sem_o, sem_s):
        w = _worker_id()
        sblk = w // 2
        b0 = (w % 2) * b_half
        s0 = sblk * SBLK

        c_idx = pltpu.async_copy(
            ip_hbm.at[pl.ds(b0, b_half), pl.ds(s0, SBLK)], idxp_v, sem_in)
        c_pe = pltpu.async_copy(pe_hbm.at[pl.ds(s0, SBLK), :], pe_v, sem_in)
        c_idx.wait()
        c_pe.wait()

        def gather_copy(c, slot):
            return pltpu.make_async_copy(
                W2_hbm.at[idxp_v.at[c]], buf_v.at[slot], sem_g.at[slot])

        def off_copy(c, slot):
            return pltpu.make_async_copy(
                off_hbm.at[b0 + c, pl.ds(s0, SBLK)], off_s.at[slot],
                sem_o.at[slot])

        def compute(slot):
            @pl.loop(0, SBLK, step=LANES)
            def _(si):
                offv = off_s[slot, pl.ds(si, LANES)]
                for u in range(LANES):
                    s = si + u
                    off = offv[u]
                    vals = [buf_v[slot, s, pl.ds(off + q * LANES, LANES)]
                            for q in range(DIM // LANES)]
                    pes = [pe_v[s, pl.ds(q * LANES, LANES)]
                           for q in range(DIM // LANES)]
                    for q in range(DIM // LANES):
                        out_v[slot, s, pl.ds(q * LANES, LANES)] = (
                            vals[q] * SQRT_DIM + pes[q])

        def writeback_copy(c, slot):
            return pltpu.make_async_copy(
                out_v.at[slot],
                o_hbm.at[b0 + c, pl.ds(s0, SBLK), :],
                sem_s.at[slot])

        gather_copy(0, 0).start()
        off_copy(0, 0).start()

        @pl.loop(0, b_half, step=NBUF)
        def _(c0):
            for u in range(NBUF):
                c = c0 + u
                slot = u
                nslot = (u + 1) % NBUF

                @pl.when(c + 1 < b_half)
                def _(c=c, nslot=nslot):
                    gather_copy(c + 1, nslot).start()
                    off_copy(c + 1, nslot).start()

                gather_copy(c, slot).wait()
                off_copy(c, slot).wait()

                @pl.when(c >= NBUF)
                def _(c=c, slot=slot):
                    writeback_copy(c - NBUF, slot).wait()

                compute(slot)
                writeback_copy(c, slot).start()

        for u in range(NBUF):
            writeback_copy(b_half - NBUF + u, u).wait()

    return gather_fn(W2, idxp_t, off_t, pe2d)


def kernel(input, W):
    S, B, _ = input.shape
    idx_t = jnp.transpose(input[..., 0])
    idxp_t = lax.shift_right_logical(idx_t, 1)
    off_t = lax.shift_left(lax.bitwise_and(idx_t, 1), 6)
    W_t = jnp.transpose(W)
    pe2d = jnp.asarray(np.ascontiguousarray(_make_pe_t(S).T))
    out_bs = _embed_sc(idxp_t, off_t, W_t, pe2d, S=S, B=B)
    return jnp.transpose(out_bs, (1, 0, 2))


# speedup vs baseline: 1.9593x; 1.8205x over previous
import math
import functools

import jax
import jax.numpy as jnp
import numpy as np
from jax import lax
from jax.experimental import pallas as pl
from jax.experimental.pallas import tpu as pltpu
from jax.experimental.pallas import tpu_sc as plsc

DIM = 64
MAX_LEN = 5000
SQRT_DIM = math.sqrt(DIM)

LANES = 16
NWORKERS = 32
STREAM_W = 128
CHUNK = 512
NBUF = 3
NVREG = DIM // LANES


def _make_pe_2d(seq_len: int) -> np.ndarray:
    position = np.arange(0, MAX_LEN, dtype=np.float64)[:, None]
    div_term = np.exp(
        np.arange(0, DIM, 2, dtype=np.float64) * -(math.log(10000.0) / DIM)
    )
    pe = np.zeros((MAX_LEN, DIM), dtype=np.float64)
    pe[:, 0::2] = np.sin(position * div_term)
    pe[:, 1::2] = np.cos(position * div_term)
    return pe[:seq_len].astype(np.float32)


@functools.partial(jax.jit, static_argnames=("S", "B"))
def _embed_sc(idx3, W, pe, *, S, B):
    N = S * B
    per_w = N // NWORKERS
    n_chunks = per_w // CHUNK
    streams_per_chunk = CHUNK // STREAM_W
    s_per_chunk = CHUNK // B
    s_per_w = per_w // B

    mesh = plsc.VectorSubcoreMesh(core_axis_name="core",
                                  subcore_axis_name="subcore")

    @pl.kernel(
        out_type=jax.ShapeDtypeStruct((N, DIM), jnp.float32),
        mesh=mesh,
        compiler_params=pltpu.CompilerParams(use_tc_tiling_on_sc=False),
        scratch_types=[
            pltpu.VMEM((per_w // STREAM_W, STREAM_W), jnp.int32),
            pltpu.VMEM((s_per_w, DIM), jnp.float32),
            pltpu.VMEM((NBUF, CHUNK, DIM), jnp.float32),
            pltpu.SemaphoreType.DMA,
            pltpu.SemaphoreType.DMA((NBUF,)),
            pltpu.SemaphoreType.DMA((NBUF,)),
        ],
    )
    def kernel_fn(W_hbm, i_hbm, pe_hbm, o_hbm,
                  idx_v, pe_v, buf_v, sem_in, sem_g, sem_s):
        wid = lax.axis_index("core") * 16 + lax.axis_index("subcore")
        row0 = wid * per_w

        c0 = pltpu.async_copy(i_hbm.at[wid], idx_v, sem_in)
        c1 = pltpu.async_copy(pe_hbm.at[pl.ds(wid * s_per_w, s_per_w)],
                              pe_v, sem_in)
        c0.wait()
        c1.wait()

        def fire_gather(j, b):
            for k in range(streams_per_chunk):
                pltpu.async_copy(
                    W_hbm.at[idx_v.at[j * streams_per_chunk + k]],
                    buf_v.at[b, pl.ds(k * STREAM_W, STREAM_W)],
                    sem_g.at[b],
                )

        def wait_gather(j, b):
            for k in range(streams_per_chunk):
                pltpu.make_async_copy(
                    W_hbm.at[idx_v.at[j * streams_per_chunk + k]],
                    buf_v.at[b, pl.ds(k * STREAM_W, STREAM_W)],
                    sem_g.at[b],
                ).wait()

        def compute(j, b):
            for g in range(s_per_chunk):
                pe_regs = [pe_v[j * s_per_chunk + g, pl.ds(q * LANES, LANES)]
                           for q in range(NVREG)]

                @pl.loop(0, B)
                def _(r, g=g, pe_regs=pe_regs):
                    row = g * B + r
                    for q in range(NVREG):
                        sl = pl.ds(q * LANES, LANES)
                        buf_v[b, row, sl] = (buf_v[b, row, sl] * SQRT_DIM
                                             + pe_regs[q])

        def scatter(j, b):
            return pltpu.async_copy(
                buf_v.at[b],
                o_hbm.at[pl.ds(row0 + j * CHUNK, CHUNK)],
                sem_s.at[b],
            )

        scatter_handles = [None] * NBUF
        fire_gather(0, 0)
        for j in range(n_chunks):
            b = j % NBUF
            wait_gather(j, b)
            if j + 1 < n_chunks:
                nb = (j + 1) % NBUF
                if scatter_handles[nb] is not None:
                    scatter_handles[nb].wait()
                    scatter_handles[nb] = None
                fire_gather(j + 1, nb)
            compute(j, b)
            scatter_handles[b] = scatter(j, b)
        for h in scatter_handles:
            if h is not None:
                h.wait()

    return kernel_fn(W, idx3, pe)


def kernel(input, W):
    S, B, _ = input.shape
    N = S * B
    idx3 = input[..., 0].reshape(NWORKERS, (N // NWORKERS) // STREAM_W, STREAM_W)
    pe = jnp.asarray(_make_pe_2d(S))
    out = _embed_sc(idx3, W, pe, S=S, B=B)
    return out.reshape(S, B, DIM)
